# baseline (speedup 1.0000x reference)
import functools

import jax
import jax.numpy as jnp
from jax import lax
from jax.experimental import pallas as pl
from jax.experimental.pallas import tpu as pltpu
from jax.experimental.pallas import tpu_sc as plsc

N = 10000
N2 = 10240
E = 320000
NC = 2
NS = 16
NW = NC * NS
CH = 128
NCH = 80
E2 = NW * NCH * CH
KC0 = 80
KC1 = 2 * NCH - KC0
KMAX = max(KC0, KC1)
NCR = E2 // CH + 96
NPT = N2 // NS
NZC = NPT // CH

_MESH = plsc.VectorSubcoreMesh(core_axis_name="c", subcore_axis_name="s")


def _fill(ref, value):
    d = ref.shape[1]
    vec = jnp.full((1, 16), value, jnp.float32)

    @pl.loop(0, CH)
    def _(i):
        for j in range(d // 16):
            ref[pl.ds(i, 1), pl.ds(j * 16, 16)] = vec


def _zero_acc_slice(rows, acc, s):
    for z in range(NZC):
        pltpu.sync_copy(rows, acc.at[pl.ds(s * NPT + z * CH, CH)])


def _copy_out_slice(acc, out_hbm, c, s):
    for z in range(NZC):
        pltpu.sync_copy(acc.at[pl.ds(s * NPT + z * CH, CH)],
                        out_hbm.at[c, pl.ds(s * NPT + z * CH, CH)])


def _make_seg_sum(d):
    @functools.partial(
        pl.kernel,
        out_type=jax.ShapeDtypeStruct((NC, N2, d), jnp.float32),
        mesh=_MESH,
        scratch_types=[
            pltpu.VMEM((NCH, CH), jnp.int32),
            pltpu.VMEM((CH,), jnp.int32),
            pltpu.VMEM((CH,), jnp.int32),
            pltpu.VMEM((CH, d), jnp.float32),
            pltpu.VMEM((CH, d), jnp.float32),
            pltpu.VMEM_SHARED((N2, d), jnp.float32),
            pltpu.SemaphoreType.DMA,
            pltpu.SemaphoreType.DMA,
            pltpu.SemaphoreType.DMA,
            pltpu.SemaphoreType.DMA,
        ],
    )
    def seg(src2d_hbm, dst2d_hbm, h_hbm, out_hbm, sidx, didx0, didx1,
            rows0, rows1, acc, sem0, sem1, semd0, semd1):
        c = lax.axis_index("c")
        s = lax.axis_index("s")
        wid = s * NC + c
        cbase = wid * NCH

        pltpu.sync_copy(src2d_hbm.at[pl.ds(cbase, NCH)], sidx)
        _fill(rows0, 0.0)
        _zero_acc_slice(rows0, acc, s)
        plsc.subcore_barrier()

        pltpu.async_copy(h_hbm.at[sidx.at[0]], rows0, sem0)
        pltpu.async_copy(dst2d_hbm.at[cbase], didx0, semd0)

        @pl.loop(0, NCH, step=2)
        def _(i):
            pltpu.make_async_copy(h_hbm.at[sidx.at[i]], rows0, sem0).wait()
            pltpu.make_async_copy(dst2d_hbm.at[cbase + i], didx0, semd0).wait()
            pltpu.async_copy(h_hbm.at[sidx.at[i + 1]], rows1, sem1)
            pltpu.async_copy(dst2d_hbm.at[cbase + i + 1], didx1, semd1)
            pltpu.sync_copy(rows0, acc.at[didx0], add=True)
            pltpu.make_async_copy(h_hbm.at[sidx.at[i + 1]], rows1, sem1).wait()
            pltpu.make_async_copy(dst2d_hbm.at[cbase + i + 1], didx1,
                                  semd1).wait()

            @pl.when(i + 2 < NCH)
            def _():
                pltpu.async_copy(h_hbm.at[sidx.at[i + 2]], rows0, sem0)
                pltpu.async_copy(dst2d_hbm.at[cbase + i + 2], didx0, semd0)

            pltpu.sync_copy(rows1, acc.at[didx1], add=True)

        plsc.subcore_barrier()
        _copy_out_slice(acc, out_hbm, c, s)

    return seg


@functools.partial(
    pl.kernel,
    out_type=jax.ShapeDtypeStruct((NC, N2, 128), jnp.float32),
    mesh=_MESH,
    scratch_types=[
        pltpu.VMEM((CH,), jnp.int32),
        pltpu.VMEM((CH, 128), jnp.float32),
        pltpu.VMEM_SHARED((N2, 128), jnp.float32),
    ],
)
def _deg_kernel(dst2d_hbm, out_hbm, didx, ones, acc):
    c = lax.axis_index("c")
    s = lax.axis_index("s")
    wid = s * NC + c
    cbase = wid * NCH

    _fill(ones, 0.0)
    _zero_acc_slice(ones, acc, s)
    plsc.subcore_barrier()
    _fill(ones, 1.0)

    @pl.loop(0, NCH)
    def _(i):
        pltpu.sync_copy(dst2d_hbm.at[cbase + i], didx)
        pltpu.sync_copy(ones, acc.at[didx], add=True)

    plsc.subcore_barrier()
    _copy_out_slice(acc, out_hbm, c, s)


_seg_sum_128 = _make_seg_sum(128)

BM = 1024


def _tc_prep(deg2, x):
    def body(deg_ref, x_ref, dinv_ref, xs_ref):
        d = (deg_ref[0] + deg_ref[1])[:, :1]
        dv = lax.rsqrt(jnp.maximum(d, 1.0))
        dinv_ref[...] = jnp.broadcast_to(dv, (BM, 16))
        xs_ref[...] = x_ref[...] * dv

    return pl.pallas_call(
        body,
        grid=(N2 // BM,),
        in_specs=[pl.BlockSpec((2, BM, 128), lambda i: (0, i, 0)),
                  pl.BlockSpec((BM, 128), lambda i: (i, 0))],
        out_specs=[pl.BlockSpec((BM, 16), lambda i: (i, 0)),
                   pl.BlockSpec((BM, 128), lambda i: (i, 0))],
        out_shape=[jax.ShapeDtypeStruct((N2, 16), jnp.float32),
                   jax.ShapeDtypeStruct((N2, 128), jnp.float32)],
    )(deg2, x)


def _tc_mid(sacc, dinv, w, b):
    def body(s_ref, dinv_ref, w_ref, b_ref, o_ref):
        dv = dinv_ref[:, :1]
        a = (s_ref[0] + s_ref[1]) * dv
        h = jnp.dot(a, w_ref[...], preferred_element_type=jnp.float32)
        h = jnp.maximum(h + b_ref[...], 0.0)
        o_ref[...] = h * dv

    return pl.pallas_call(
        body,
        grid=(N2 // BM,),
        in_specs=[pl.BlockSpec((2, BM, 128), lambda i: (0, i, 0)),
                  pl.BlockSpec((BM, 16), lambda i: (i, 0)),
                  pl.BlockSpec((128, 128), lambda i: (0, 0)),
                  pl.BlockSpec((1, 128), lambda i: (0, 0))],
        out_specs=pl.BlockSpec((BM, 128), lambda i: (i, 0)),
        out_shape=jax.ShapeDtypeStruct((N2, 128), jnp.float32),
    )(sacc, dinv, w, b)


def _tc_last(sacc, dinv, w1, b1, w2p):
    def body(s_ref, dinv_ref, w1_ref, b1_ref, w2_ref, o_ref):
        dv = dinv_ref[:, :1]
        a = (s_ref[0] + s_ref[1]) * dv
        h = jnp.dot(a, w1_ref[...], preferred_element_type=jnp.float32)
        h = jnp.maximum(h + b1_ref[...], 0.0)
        g = jnp.dot(h, w2_ref[...], preferred_element_type=jnp.float32)
        o_ref[...] = g * dv

    return pl.pallas_call(
        body,
        grid=(N2 // BM,),
        in_specs=[pl.BlockSpec((2, BM, 128), lambda i: (0, i, 0)),
                  pl.BlockSpec((BM, 16), lambda i: (i, 0)),
                  pl.BlockSpec((128, 128), lambda i: (0, 0)),
                  pl.BlockSpec((1, 128), lambda i: (0, 0)),
                  pl.BlockSpec((128, 128), lambda i: (0, 0))],
        out_specs=pl.BlockSpec((BM, 128), lambda i: (i, 0)),
        out_shape=jax.ShapeDtypeStruct((N2, 128), jnp.float32),
    )(sacc, dinv, w1, b1, w2p)


def _tc_final(sacc, dinv, b):
    def body(s_ref, dinv_ref, b_ref, o_ref):
        t = (s_ref[0] + s_ref[1])[:, :16]
        o_ref[...] = t * dinv_ref[:, :1] + b_ref[...]

    return pl.pallas_call(
        body,
        grid=(N2 // BM,),
        in_specs=[pl.BlockSpec((2, BM, 128), lambda i: (0, i, 0)),
                  pl.BlockSpec((BM, 16), lambda i: (i, 0)),
                  pl.BlockSpec((1, 16), lambda i: (0, 0))],
        out_specs=pl.BlockSpec((BM, 16), lambda i: (i, 0)),
        out_shape=jax.ShapeDtypeStruct((N2, 16), jnp.float32),
    )(sacc, dinv, b)


def kernel(x, edge_index, W0, b0, W1, b1, W2, b2):
    sink = N + (jnp.arange(E2 - E, dtype=jnp.int32) % (N2 - N))
    rpad = ((0, NCR - E2 // CH), (0, 0))
    src2d = jnp.pad(jnp.concatenate(
        [edge_index[0],
         jnp.zeros((E2 - E,), jnp.int32)]).reshape(E2 // CH, CH), rpad)
    dst2d = jnp.pad(jnp.concatenate(
        [edge_index[1], sink]).reshape(E2 // CH, CH), rpad)
    xp = jnp.pad(x, ((0, N2 - N), (0, 0)))
    deg2 = _deg_kernel(dst2d)
    dinv, xs = _tc_prep(deg2, xp)
    s0 = _seg_sum_128(src2d, dst2d, xs)
    hs1 = _tc_mid(s0, dinv, W0, b0.reshape(1, 128))
    s1 = _seg_sum_128(src2d, dst2d, hs1)
    w2p = jnp.pad(W2, ((0, 0), (0, 128 - W2.shape[1])))
    g = _tc_last(s1, dinv, W1, b1.reshape(1, 128), w2p)
    s2 = _seg_sum_128(src2d, dst2d, g)
    out = _tc_final(s2, dinv, b2.reshape(1, 16))
    return out[:N]


# speedup vs baseline: 1.3392x; 1.3392x over previous
import functools

import jax
import jax.numpy as jnp
from jax import lax
from jax.experimental import pallas as pl
from jax.experimental.pallas import tpu as pltpu
from jax.experimental.pallas import tpu_sc as plsc

N = 10000
N2 = 10240
E = 320000
NC = 2
NS = 16
NW = NC * NS
CH = 128
NCH = 80
E2 = NW * NCH * CH
KC0 = 80
KC1 = 2 * NCH - KC0
KMAX = max(KC0, KC1)
NCR = E2 // CH + 96
NPT = N2 // NS
NZC = NPT // CH

_MESH = plsc.VectorSubcoreMesh(core_axis_name="c", subcore_axis_name="s")


def _fill(ref, value):
    d = ref.shape[1]
    vec = jnp.full((1, 16), value, jnp.float32)

    @pl.loop(0, CH)
    def _(i):
        for j in range(d // 16):
            ref[pl.ds(i, 1), pl.ds(j * 16, 16)] = vec


def _zero_acc_slice(rows, acc, s):
    for z in range(NZC):
        pltpu.sync_copy(rows, acc.at[pl.ds(s * NPT + z * CH, CH)])


def _copy_out_slice(acc, out_hbm, c, s):
    for z in range(NZC):
        pltpu.sync_copy(acc.at[pl.ds(s * NPT + z * CH, CH)],
                        out_hbm.at[c, pl.ds(s * NPT + z * CH, CH)])


def _make_seg_sum(d):
    @functools.partial(
        pl.kernel,
        out_type=jax.ShapeDtypeStruct((NC, N2, d), jnp.float32),
        mesh=_MESH,
        scratch_types=[
            pltpu.VMEM((NCH, CH), jnp.int32),
            pltpu.VMEM((CH,), jnp.int32),
            pltpu.VMEM((CH,), jnp.int32),
            pltpu.VMEM((CH, d), jnp.float32),
            pltpu.VMEM((CH, d), jnp.float32),
            pltpu.VMEM_SHARED((N2, d), jnp.float32),
            pltpu.SemaphoreType.DMA,
            pltpu.SemaphoreType.DMA,
            pltpu.SemaphoreType.DMA,
            pltpu.SemaphoreType.DMA,
        ],
    )
    def seg(src2d_hbm, dst2d_hbm, h_hbm, out_hbm, sidx, didx0, didx1,
            rows0, rows1, acc, sem0, sem1, semd0, semd1):
        c = lax.axis_index("c")
        s = lax.axis_index("s")
        wid = s * NC + c
        cbase = wid * NCH

        pltpu.sync_copy(src2d_hbm.at[pl.ds(cbase, NCH)], sidx)
        _fill(rows0, 0.0)
        _zero_acc_slice(rows0, acc, s)
        plsc.subcore_barrier()

        pltpu.async_copy(h_hbm.at[sidx.at[0]], rows0, sem0)
        pltpu.async_copy(dst2d_hbm.at[cbase], didx0, semd0)

        @pl.loop(0, NCH, step=2)
        def _(i):
            pltpu.make_async_copy(h_hbm.at[sidx.at[i]], rows0, sem0).wait()
            pltpu.make_async_copy(dst2d_hbm.at[cbase + i], didx0, semd0).wait()
            pltpu.async_copy(h_hbm.at[sidx.at[i + 1]], rows1, sem1)
            pltpu.async_copy(dst2d_hbm.at[cbase + i + 1], didx1, semd1)
            pltpu.sync_copy(rows0, acc.at[didx0], add=True)
            pltpu.make_async_copy(h_hbm.at[sidx.at[i + 1]], rows1, sem1).wait()
            pltpu.make_async_copy(dst2d_hbm.at[cbase + i + 1], didx1,
                                  semd1).wait()

            @pl.when(i + 2 < NCH)
            def _():
                pltpu.async_copy(h_hbm.at[sidx.at[i + 2]], rows0, sem0)
                pltpu.async_copy(dst2d_hbm.at[cbase + i + 2], didx0, semd0)

            pltpu.sync_copy(rows1, acc.at[didx1], add=True)

        plsc.subcore_barrier()
        _copy_out_slice(acc, out_hbm, c, s)

    return seg


@functools.partial(
    pl.kernel,
    out_type=jax.ShapeDtypeStruct((NC, N2, 128), jnp.float32),
    mesh=_MESH,
    scratch_types=[
        pltpu.VMEM((CH,), jnp.int32),
        pltpu.VMEM((CH, 128), jnp.float32),
        pltpu.VMEM_SHARED((N2, 128), jnp.float32),
    ],
)
def _deg_kernel(dst2d_hbm, out_hbm, didx, ones, acc):
    c = lax.axis_index("c")
    s = lax.axis_index("s")
    wid = s * NC + c
    cbase = wid * NCH

    _fill(ones, 0.0)
    _zero_acc_slice(ones, acc, s)
    plsc.subcore_barrier()
    _fill(ones, 1.0)

    @pl.loop(0, NCH)
    def _(i):
        pltpu.sync_copy(dst2d_hbm.at[cbase + i], didx)
        pltpu.sync_copy(ones, acc.at[didx], add=True)

    plsc.subcore_barrier()
    _copy_out_slice(acc, out_hbm, c, s)


_seg_sum_128 = _make_seg_sum(128)

BM = 1024


def _tc_prep(deg2, x):
    def body(deg_ref, x_ref, dinv_ref, xs_ref):
        d = (deg_ref[0] + deg_ref[1])[:, :1]
        dv = lax.rsqrt(jnp.maximum(d, 1.0))
        dinv_ref[...] = jnp.broadcast_to(dv, (BM, 16))
        xs_ref[...] = x_ref[...] * dv

    return pl.pallas_call(
        body,
        grid=(N2 // BM,),
        in_specs=[pl.BlockSpec((2, BM, 128), lambda i: (0, i, 0)),
                  pl.BlockSpec((BM, 128), lambda i: (i, 0))],
        out_specs=[pl.BlockSpec((BM, 16), lambda i: (i, 0)),
                   pl.BlockSpec((BM, 128), lambda i: (i, 0))],
        out_shape=[jax.ShapeDtypeStruct((N2, 16), jnp.float32),
                   jax.ShapeDtypeStruct((N2, 128), jnp.float32)],
    )(deg2, x)


def _tc_mid(sacc, dinv, w, b):
    def body(s_ref, dinv_ref, w_ref, b_ref, o_ref):
        dv = dinv_ref[:, :1]
        a = (s_ref[0] + s_ref[1]) * dv
        h = jnp.dot(a, w_ref[...], preferred_element_type=jnp.float32)
        h = jnp.maximum(h + b_ref[...], 0.0)
        o_ref[...] = h * dv

    return pl.pallas_call(
        body,
        grid=(N2 // BM,),
        in_specs=[pl.BlockSpec((2, BM, 128), lambda i: (0, i, 0)),
                  pl.BlockSpec((BM, 16), lambda i: (i, 0)),
                  pl.BlockSpec((128, 128), lambda i: (0, 0)),
                  pl.BlockSpec((1, 128), lambda i: (0, 0))],
        out_specs=pl.BlockSpec((BM, 128), lambda i: (i, 0)),
        out_shape=jax.ShapeDtypeStruct((N2, 128), jnp.float32),
    )(sacc, dinv, w, b)


def _tc_last(sacc, dinv, w1, b1, w2p):
    def body(s_ref, dinv_ref, w1_ref, b1_ref, w2_ref, o_ref):
        dv = dinv_ref[:, :1]
        a = (s_ref[0] + s_ref[1]) * dv
        h = jnp.dot(a, w1_ref[...], preferred_element_type=jnp.float32)
        h = jnp.maximum(h + b1_ref[...], 0.0)
        g = jnp.dot(h, w2_ref[...], preferred_element_type=jnp.float32)
        o_ref[...] = g * dv

    return pl.pallas_call(
        body,
        grid=(N2 // BM,),
        in_specs=[pl.BlockSpec((2, BM, 128), lambda i: (0, i, 0)),
                  pl.BlockSpec((BM, 16), lambda i: (i, 0)),
                  pl.BlockSpec((128, 128), lambda i: (0, 0)),
                  pl.BlockSpec((1, 128), lambda i: (0, 0)),
                  pl.BlockSpec((128, 128), lambda i: (0, 0))],
        out_specs=pl.BlockSpec((BM, 128), lambda i: (i, 0)),
        out_shape=jax.ShapeDtypeStruct((N2, 128), jnp.float32),
    )(sacc, dinv, w1, b1, w2p)


def _tc_final(sacc, dinv, b):
    def body(s_ref, dinv_ref, b_ref, o_ref):
        t = (s_ref[0] + s_ref[1])[:, :16]
        o_ref[...] = t * dinv_ref[:, :1] + b_ref[...]

    return pl.pallas_call(
        body,
        grid=(N2 // BM,),
        in_specs=[pl.BlockSpec((2, BM, 128), lambda i: (0, i, 0)),
                  pl.BlockSpec((BM, 16), lambda i: (i, 0)),
                  pl.BlockSpec((1, 16), lambda i: (0, 0))],
        out_specs=pl.BlockSpec((BM, 16), lambda i: (i, 0)),
        out_shape=jax.ShapeDtypeStruct((N2, 16), jnp.float32),
    )(sacc, dinv, b)


def kernel(x, edge_index, W0, b0, W1, b1, W2, b2):
    sink = N + (jnp.arange(E2 - E, dtype=jnp.int32) % (N2 - N))
    src2d = jnp.concatenate(
        [edge_index[0], jnp.zeros((E2 - E,), jnp.int32)]).reshape(E2 // CH, CH)
    dst2d = jnp.concatenate(
        [edge_index[1], sink]).reshape(E2 // CH, CH)
    xp = jnp.pad(x, ((0, N2 - N), (0, 0)))
    deg2 = _deg_kernel(dst2d)
    dinv, xs = _tc_prep(deg2, xp)
    s0 = _seg_sum_128(src2d, dst2d, xs)
    hs1 = _tc_mid(s0, dinv, W0, b0.reshape(1, 128))
    s1 = _seg_sum_128(src2d, dst2d, hs1)
    w2p = jnp.pad(W2, ((0, 0), (0, 128 - W2.shape[1])))
    g = _tc_last(s1, dinv, W1, b1.reshape(1, 128), w2p)
    s2 = _seg_sum_128(src2d, dst2d, g)
    out = _tc_final(s2, dinv, b2.reshape(1, 16))
    return out[:N]
